# Initial kernel scaffold; baseline (speedup 1.0000x reference)
import jax
import jax.numpy as jnp
from jax.experimental import pallas as pl


def kernel(x, Wf, bf, Wv, bv, Ws, bs, gw, gb):
    raise NotImplementedError("write your pallas kernel here")


# speedup vs baseline: 2.8717x; 2.8717x over previous
import functools

import jax
import jax.numpy as jnp
from jax.experimental import pallas as pl

_EPS = 1e-12
_FOLD = 8


def _tile_kernel(xt_ref, wf_ref, bf_ref, wv_ref, bv_ref, pool_ref, pooled_ref,
                 stats_ref, *, M, N, P, Q):
    xp = xt_ref[0]
    xf = jnp.dot(wf_ref[...], xp, preferred_element_type=jnp.float32) + bf_ref[...]
    val = jnp.dot(wv_ref[...], xp, preferred_element_type=jnp.float32) + bv_ref[...]
    pmat = pool_ref[...]
    centers = jnp.dot(xf, pmat, preferred_element_type=jnp.float32)
    cn = centers / jnp.maximum(
        jnp.sqrt(jnp.sum(centers * centers, axis=0, keepdims=True)), _EPS)
    xn = xf / jnp.maximum(
        jnp.sqrt(jnp.sum(xf * xf, axis=0, keepdims=True)), _EPS)
    simt = jax.lax.dot_general(xn, cn, (((0,), (0,)), ((), ())),
                               preferred_element_type=jnp.float32)
    maxv = jnp.max(simt, axis=1, keepdims=True)
    mi = jax.lax.broadcasted_iota(jnp.int32, (N, M), 1)
    first = jnp.min(jnp.where(simt >= maxv, mi, M), axis=1, keepdims=True)
    onehot = (mi == first).astype(jnp.float32)
    counts = jnp.dot(jnp.ones((1, N), jnp.float32), onehot,
                     preferred_element_type=jnp.float32)
    scat = jnp.dot(val, onehot + pmat, preferred_element_type=jnp.float32)
    out_t = scat / (counts + 1.0)
    pooled_ref[0] = out_t
    s = jnp.sum(out_t)
    sq = jnp.sum(out_t * out_t)
    i8 = jax.lax.broadcasted_iota(jnp.int32, (1, 8), 1)
    stats_ref[0] = jnp.where(i8 == 0, s, jnp.where(i8 == 1, sq, 0.0))


def _conv_kernel(ph_ref, w_ref, b_ref, out_ref, *, RB, Wo):
    Cout = w_ref.shape[2]
    acc = None
    for dy in range(3):
        for dx in range(3):
            py, px = dy % 2, dx % 2
            ro = 1 if dy == 2 else 0
            co = 1 if dx == 2 else 0
            xt = ph_ref[0, 0, py, px]
            xs = xt[:, ro:ro + RB, co:co + Wo]
            xs = xs.reshape(xs.shape[0], RB * Wo)
            t = jnp.dot(w_ref[dy, dx], xs, preferred_element_type=jnp.float32)
            acc = t if acc is None else acc + t
    out = acc + b_ref[...]
    out_ref[0] = out.reshape(Cout, RB, Wo)


def _final_kernel(pooled_ref, id_ref, stats_ref, gw_ref, gb_ref, out_ref, *, count):
    s = jnp.sum(stats_ref[0, :, 0])
    sq = jnp.sum(stats_ref[0, :, 1])
    mu = s / count
    var = sq / count - mu * mu
    inv = jax.lax.rsqrt(var + 1e-5)
    g = gw_ref[...].reshape(-1, 1, 1)
    bb = gb_ref[...].reshape(-1, 1, 1)
    out_ref[0] = (pooled_ref[0] - mu) * inv * g + bb + id_ref[0]


def kernel(x, Wf, bf, Wv, bv, Ws, bs, gw, gb):
    B0, Cin, H, W = x.shape
    Cout = Wf.shape[0]
    f = _FOLD
    Q = H // f
    P = Q // 2
    M = P * P
    N = Q * Q
    T = B0 * f * f
    Ho = H // 2
    Hp = Ho + 1

    xt = x.reshape(B0, Cin, f, Q, f, Q).transpose(0, 2, 4, 1, 3, 5).reshape(T, Cin, N)
    wf2 = Wf.reshape(Cout, Cin)
    wv2 = Wv.reshape(Cout, Cin)
    nidx = jnp.arange(N, dtype=jnp.int32)
    mtarget = (nidx // Q // 2) * P + (nidx % Q) // 2
    pmat = (jax.nn.one_hot(mtarget, M, dtype=jnp.float32) * 0.25)
    pooled, stats = pl.pallas_call(
        functools.partial(_tile_kernel, M=M, N=N, P=P, Q=Q),
        grid=(T,),
        in_specs=[
            pl.BlockSpec((1, Cin, N), lambda t: (t, 0, 0)),
            pl.BlockSpec((Cout, Cin), lambda t: (0, 0)),
            pl.BlockSpec((Cout, 1), lambda t: (0, 0)),
            pl.BlockSpec((Cout, Cin), lambda t: (0, 0)),
            pl.BlockSpec((Cout, 1), lambda t: (0, 0)),
            pl.BlockSpec((N, M), lambda t: (0, 0)),
        ],
        out_specs=[
            pl.BlockSpec((1, Cout, M), lambda t: (t, 0, 0)),
            pl.BlockSpec((1, 1, 8), lambda t: (t, 0, 0)),
        ],
        out_shape=[
            jax.ShapeDtypeStruct((T, Cout, M), jnp.float32),
            jax.ShapeDtypeStruct((T, 1, 8), jnp.float32),
        ],
    )(xt, wf2, bf.reshape(Cout, 1), wv2, bv.reshape(Cout, 1), pmat)

    RB = 16
    nblk = Ho // RB
    xpad = jnp.pad(x, ((0, 0), (0, 0), (1, 1), (1, 1)))
    ph = xpad.reshape(B0, Cin, Hp, 2, Hp, 2).transpose(0, 3, 5, 1, 2, 4)
    phb = jnp.stack([ph[:, :, :, :, i * RB:i * RB + RB + 1, :] for i in range(nblk)],
                    axis=1)
    ws2 = Ws.transpose(2, 3, 0, 1)
    identity = pl.pallas_call(
        functools.partial(_conv_kernel, RB=RB, Wo=Ho),
        grid=(B0, nblk),
        in_specs=[
            pl.BlockSpec((1, 1, 2, 2, Cin, RB + 1, Hp), lambda b, r: (b, r, 0, 0, 0, 0, 0)),
            pl.BlockSpec((3, 3, Cout, Cin), lambda b, r: (0, 0, 0, 0)),
            pl.BlockSpec((Cout, 1), lambda b, r: (0, 0)),
        ],
        out_specs=pl.BlockSpec((1, Cout, RB, Ho), lambda b, r: (b, 0, r, 0)),
        out_shape=jax.ShapeDtypeStruct((B0, Cout, Ho, Ho), jnp.float32),
    )(phb, ws2, bs.reshape(Cout, 1))

    pooled_full = pooled.reshape(B0, f, f, Cout, P, P).transpose(0, 3, 1, 4, 2, 5)
    pooled_full = pooled_full.reshape(B0, Cout, Ho, Ho)
    stats2 = stats.reshape(B0, f * f, 8)
    cnt = float(Cout * Ho * Ho)
    RB2 = 16
    out = pl.pallas_call(
        functools.partial(_final_kernel, count=cnt),
        grid=(B0, Ho // RB2),
        in_specs=[
            pl.BlockSpec((1, Cout, RB2, Ho), lambda b, r: (b, 0, r, 0)),
            pl.BlockSpec((1, Cout, RB2, Ho), lambda b, r: (b, 0, r, 0)),
            pl.BlockSpec((1, f * f, 8), lambda b, r: (b, 0, 0)),
            pl.BlockSpec((Cout, 1), lambda b, r: (0, 0)),
            pl.BlockSpec((Cout, 1), lambda b, r: (0, 0)),
        ],
        out_specs=pl.BlockSpec((1, Cout, RB2, Ho), lambda b, r: (b, 0, r, 0)),
        out_shape=jax.ShapeDtypeStruct((B0, Cout, Ho, Ho), jnp.float32),
    )(pooled_full, identity, stats2, gw.reshape(Cout, 1), gb.reshape(Cout, 1))
    return out
